# Initial kernel scaffold; baseline (speedup 1.0000x reference)
import functools

import jax
import jax.numpy as jnp
from jax import lax
from jax.experimental import pallas as pl
from jax.experimental.pallas import tpu as pltpu
from jax.experimental.pallas import tpu_sc as plsc

N = 10000
E = 320000
D = 128

NC = 2
NS = 16
NW = NC * NS
EPW = E // NW
K = 80
NCHUNK = EPW // K
GRP = K // 16

_mesh = plsc.VectorSubcoreMesh(core_axis_name="c", subcore_axis_name="s")


def _sc_deg_body(dst_hbm, ew_hbm, zeros_hbm, degp_hbm, dst_v, ew_v, deg_sh):
    c = lax.axis_index("c")
    s = lax.axis_index("s")
    wid = s * NC + c

    @pl.when(s == 0)
    def _init():
        pltpu.sync_copy(zeros_hbm, deg_sh)

    plsc.subcore_barrier()
    base = wid * EPW

    @pl.loop(0, NCHUNK)
    def _chunk(i):
        off = base + i * K
        pltpu.sync_copy(dst_hbm.at[pl.ds(off, K)], dst_v)
        pltpu.sync_copy(ew_hbm.at[pl.ds(off, K)], ew_v)
        pltpu.sync_copy(ew_v, deg_sh.at[dst_v], add=True)

    plsc.subcore_barrier()

    @pl.when(s == 0)
    def _flush():
        pltpu.sync_copy(deg_sh, degp_hbm.at[c])


_sc_deg = pl.kernel(
    _sc_deg_body,
    out_type=jax.ShapeDtypeStruct((NC, N), jnp.float32),
    mesh=_mesh,
    scratch_types=[
        pltpu.VMEM((K,), jnp.int32),
        pltpu.VMEM((K,), jnp.float32),
        pltpu.VMEM_SHARED((N,), jnp.float32),
    ],
)


def _sc_edge_body(g_hbm, src_hbm, dst_hbm, ew_hbm, zeros_hbm, accp_hbm,
                  src_v, dst_v, ew_v, rows_v, sem, acc_sh):
    c = lax.axis_index("c")
    s = lax.axis_index("s")
    wid = s * NC + c

    @pl.when(s == 0)
    def _init():
        pltpu.sync_copy(zeros_hbm, acc_sh)

    plsc.subcore_barrier()
    base = wid * EPW
    lane = lax.iota(jnp.int32, 16)

    @pl.loop(0, NCHUNK)
    def _chunk(i):
        off = base + i * K
        pltpu.sync_copy(src_hbm.at[pl.ds(off, K)], src_v)
        pltpu.sync_copy(dst_hbm.at[pl.ds(off, K)], dst_v)
        pltpu.sync_copy(ew_hbm.at[pl.ds(off, K)], ew_v)
        pltpu.async_copy(g_hbm.at[src_v], rows_v, sem).wait()

        @pl.loop(0, GRP)
        def _grp(gi):
            e0 = gi * 16
            ew16 = ew_v[pl.ds(e0, 16)]
            ridx = e0 + lane
            for col in range(D):
                cidx = jnp.full((16,), col, jnp.int32)
                v = plsc.load_gather(rows_v, [ridx, cidx])
                plsc.store_scatter(rows_v, [ridx, cidx], v * ew16)

        pltpu.sync_copy(rows_v, acc_sh.at[dst_v], add=True)

    plsc.subcore_barrier()

    @pl.when(s == 0)
    def _flush():
        pltpu.sync_copy(acc_sh, accp_hbm.at[c])


_sc_edge = pl.kernel(
    _sc_edge_body,
    out_type=jax.ShapeDtypeStruct((NC, N, D), jnp.float32),
    mesh=_mesh,
    scratch_types=[
        pltpu.VMEM((K,), jnp.int32),
        pltpu.VMEM((K,), jnp.int32),
        pltpu.VMEM((K,), jnp.float32),
        pltpu.VMEM((K, D), jnp.float32),
        pltpu.SemaphoreType.DMA,
        pltpu.VMEM_SHARED((N, D), jnp.float32),
    ],
)

BR = 400


def _tc_lin_body(x_ref, w_ref, degp_ref, g_ref):
    deg = degp_ref[0, :] + degp_ref[1, :] + 2.0
    dis = lax.rsqrt(deg)
    h = jnp.dot(x_ref[...], w_ref[...], preferred_element_type=jnp.float32)
    g_ref[...] = h * dis[:, None]


def _tc_lin(x, w, degp):
    return pl.pallas_call(
        _tc_lin_body,
        grid=(N // BR,),
        in_specs=[
            pl.BlockSpec((BR, D), lambda i: (i, 0)),
            pl.BlockSpec((D, D), lambda i: (0, 0)),
            pl.BlockSpec((2, BR), lambda i: (0, i)),
        ],
        out_specs=pl.BlockSpec((BR, D), lambda i: (i, 0)),
        out_shape=jax.ShapeDtypeStruct((N, D), jnp.float32),
    )(x, w, degp)


def _tc_comb_body(accp_ref, g_ref, degp_ref, b_ref, o_ref):
    deg = degp_ref[0, :] + degp_ref[1, :] + 2.0
    dis = lax.rsqrt(deg)
    t = accp_ref[0] + accp_ref[1] + 2.0 * g_ref[...]
    o_ref[...] = t * dis[:, None] + b_ref[...]


def _tc_comb(accp, g, degp, b2):
    return pl.pallas_call(
        _tc_comb_body,
        grid=(N // BR,),
        in_specs=[
            pl.BlockSpec((2, BR, D), lambda i: (0, i, 0)),
            pl.BlockSpec((BR, D), lambda i: (i, 0)),
            pl.BlockSpec((2, BR), lambda i: (0, i)),
            pl.BlockSpec((1, D), lambda i: (0, 0)),
        ],
        out_specs=pl.BlockSpec((BR, D), lambda i: (i, 0)),
        out_shape=jax.ShapeDtypeStruct((N, D), jnp.float32),
    )(accp, g, degp, b2)


@jax.jit
def kernel(x, edge_index, edge_weight, W, b):
    src = edge_index[0].astype(jnp.int32)
    dst = edge_index[1].astype(jnp.int32)
    ew = edge_weight.astype(jnp.float32)
    zeros1 = jnp.zeros((N,), jnp.float32)
    zeros2 = jnp.zeros((N, D), jnp.float32)
    degp = _sc_deg(dst, ew, zeros1)
    g = _tc_lin(x, W, degp)
    accp = _sc_edge(g, src, dst, ew, zeros2)
    return _tc_comb(accp, g, degp, b.reshape(1, D))


# speedup vs baseline: 3.8033x; 3.8033x over previous
import functools

import jax
import jax.numpy as jnp
from jax import lax
from jax.experimental import pallas as pl
from jax.experimental.pallas import tpu as pltpu
from jax.experimental.pallas import tpu_sc as plsc

N = 10000
E = 320000
D = 128

NC = 2
NS = 16
NW = NC * NS
EPW = E // NW
K = 80
NCHUNK = EPW // K
GRP = K // 16

_mesh = plsc.VectorSubcoreMesh(
    core_axis_name="c", subcore_axis_name="s", num_cores=NC, num_subcores=NS
)


def _sc_deg_body(dst_hbm, ew_hbm, zeros_hbm, degp_hbm, dst_v, ew_v, deg_sh):
    c = lax.axis_index("c")
    s = lax.axis_index("s")
    wid = s * NC + c

    @pl.when(s == 0)
    def _init():
        pltpu.sync_copy(zeros_hbm, deg_sh)

    plsc.subcore_barrier()
    base = wid * EPW

    @pl.loop(0, NCHUNK)
    def _chunk(i):
        off = base + i * K
        pltpu.sync_copy(dst_hbm.at[pl.ds(off, K)], dst_v)
        pltpu.sync_copy(ew_hbm.at[pl.ds(off, K)], ew_v)
        pltpu.sync_copy(ew_v, deg_sh.at[dst_v], add=True)

    plsc.subcore_barrier()

    @pl.when(s == 0)
    def _flush():
        pltpu.sync_copy(deg_sh, degp_hbm.at[c])


_sc_deg = pl.kernel(
    _sc_deg_body,
    out_type=jax.ShapeDtypeStruct((NC, N), jnp.float32),
    mesh=_mesh,
    scratch_types=[
        pltpu.VMEM((K,), jnp.int32),
        pltpu.VMEM((K,), jnp.float32),
        pltpu.VMEM_SHARED((N,), jnp.float32),
    ],
    compiler_params=pltpu.CompilerParams(needs_layout_passes=False),
)


def _sc_edge_body(g_hbm, src_hbm, dst_hbm, ew_hbm, zeros_hbm, accp_hbm,
                  src_v, dst_v, ew_v, rows_v, sem, acc_sh):
    c = lax.axis_index("c")
    s = lax.axis_index("s")
    wid = s * NC + c

    @pl.when(s == 0)
    def _init():
        pltpu.sync_copy(zeros_hbm, acc_sh)

    plsc.subcore_barrier()
    base = wid * EPW
    lane = lax.iota(jnp.int32, 16)

    @pl.loop(0, NCHUNK)
    def _chunk(i):
        off = base + i * K
        pltpu.sync_copy(src_hbm.at[pl.ds(off, K)], src_v)
        pltpu.sync_copy(dst_hbm.at[pl.ds(off, K)], dst_v)
        pltpu.sync_copy(ew_hbm.at[pl.ds(off, K)], ew_v)
        pltpu.async_copy(g_hbm.at[src_v], rows_v, sem).wait()

        @pl.loop(0, GRP)
        def _grp(gi):
            e0 = gi * 16
            ew16 = ew_v[pl.ds(e0, 16)]
            ridx = e0 + lane
            for col in range(D):
                cidx = jnp.full((16,), col, jnp.int32)
                v = plsc.load_gather(rows_v, [ridx, cidx])
                plsc.store_scatter(rows_v, [ridx, cidx], v * ew16)

        pltpu.sync_copy(rows_v, acc_sh.at[dst_v], add=True)

    plsc.subcore_barrier()

    @pl.when(s == 0)
    def _flush():
        pltpu.sync_copy(acc_sh, accp_hbm.at[c])


_sc_edge = pl.kernel(
    _sc_edge_body,
    out_type=jax.ShapeDtypeStruct((NC, N, D), jnp.float32),
    mesh=_mesh,
    scratch_types=[
        pltpu.VMEM((K,), jnp.int32),
        pltpu.VMEM((K,), jnp.int32),
        pltpu.VMEM((K,), jnp.float32),
        pltpu.VMEM((K, D), jnp.float32),
        pltpu.SemaphoreType.DMA,
        pltpu.VMEM_SHARED((N, D), jnp.float32),
    ],
    compiler_params=pltpu.CompilerParams(needs_layout_passes=False),
)

BR = 512


def _tc_lin_body(x_ref, w_ref, degp_ref, g_ref):
    deg = degp_ref[0, :] + degp_ref[1, :] + 2.0
    dis = lax.rsqrt(deg)
    h = jnp.dot(x_ref[...], w_ref[...], preferred_element_type=jnp.float32)
    g_ref[...] = h * dis[:, None]


def _tc_lin(x, w, degp):
    return pl.pallas_call(
        _tc_lin_body,
        grid=(pl.cdiv(N, BR),),
        in_specs=[
            pl.BlockSpec((BR, D), lambda i: (i, 0)),
            pl.BlockSpec((D, D), lambda i: (0, 0)),
            pl.BlockSpec((2, BR), lambda i: (0, i)),
        ],
        out_specs=pl.BlockSpec((BR, D), lambda i: (i, 0)),
        out_shape=jax.ShapeDtypeStruct((N, D), jnp.float32),
    )(x, w, degp)


def _tc_comb_body(accp_ref, g_ref, degp_ref, b_ref, o_ref):
    deg = degp_ref[0, :] + degp_ref[1, :] + 2.0
    dis = lax.rsqrt(deg)
    t = accp_ref[0] + accp_ref[1] + 2.0 * g_ref[...]
    o_ref[...] = t * dis[:, None] + b_ref[...]


def _tc_comb(accp, g, degp, b2):
    return pl.pallas_call(
        _tc_comb_body,
        grid=(pl.cdiv(N, BR),),
        in_specs=[
            pl.BlockSpec((2, BR, D), lambda i: (0, i, 0)),
            pl.BlockSpec((BR, D), lambda i: (i, 0)),
            pl.BlockSpec((2, BR), lambda i: (0, i)),
            pl.BlockSpec((1, D), lambda i: (0, 0)),
        ],
        out_specs=pl.BlockSpec((BR, D), lambda i: (i, 0)),
        out_shape=jax.ShapeDtypeStruct((N, D), jnp.float32),
    )(accp, g, degp, b2)


@jax.jit
def kernel(x, edge_index, edge_weight, W, b):
    src = edge_index[0].astype(jnp.int32)
    dst = edge_index[1].astype(jnp.int32)
    ew = edge_weight.astype(jnp.float32)
    zeros1 = jnp.zeros((N,), jnp.float32)
    zeros2 = jnp.zeros((N, D), jnp.float32)
    degp = _sc_deg(dst, ew, zeros1)
    g = _tc_lin(x, W, degp)
    accp = _sc_edge(g, src, dst, ew, zeros2)
    return _tc_comb(accp, g, degp, b.reshape(1, D))


# speedup vs baseline: 3.9838x; 1.0475x over previous
import jax
import jax.numpy as jnp
from jax import lax
from jax.experimental import pallas as pl
from jax.experimental.pallas import tpu as pltpu
from jax.experimental.pallas import tpu_sc as plsc

N = 10000
E = 320000
D = 128

NC = 2
NS = 16
NW = NC * NS
EPW = E // NW
K = 80
NCHUNK = EPW // K
GRP = K // 16

_mesh = plsc.VectorSubcoreMesh(
    core_axis_name="c", subcore_axis_name="s", num_cores=NC, num_subcores=NS
)
_sc_params = pltpu.CompilerParams(needs_layout_passes=False)


def _sc_deg_body(dst3, ew3, zeros_hbm, degp_hbm, dsts_v, ew_v, sem, deg_sh):
    c = lax.axis_index("c")
    s = lax.axis_index("s")
    wid = s * NC + c

    @pl.when(s == 0)
    def _init():
        pltpu.sync_copy(zeros_hbm, deg_sh)

    pltpu.sync_copy(dst3.at[wid], dsts_v)
    pltpu.sync_copy(ew3.at[wid], ew_v)
    plsc.subcore_barrier()

    @pl.loop(0, NCHUNK)
    def _fire(ci):
        pltpu.async_copy(ew_v.at[ci], deg_sh.at[dsts_v.at[ci]], sem, add=True)

    @pl.loop(0, NCHUNK)
    def _drain(ci):
        pltpu.make_async_copy(ew_v.at[ci], deg_sh.at[dsts_v.at[ci]], sem).wait()

    plsc.subcore_barrier()

    @pl.when(s == 0)
    def _flush():
        pltpu.sync_copy(deg_sh, degp_hbm.at[c])


_sc_deg = pl.kernel(
    _sc_deg_body,
    out_type=jax.ShapeDtypeStruct((NC, N), jnp.float32),
    mesh=_mesh,
    scratch_types=[
        pltpu.VMEM((NCHUNK, K), jnp.int32),
        pltpu.VMEM((NCHUNK, K), jnp.float32),
        pltpu.SemaphoreType.DMA,
        pltpu.VMEM_SHARED((N,), jnp.float32),
    ],
    compiler_params=_sc_params,
)


def _sc_edge_body(g_hbm, src3, dst3, ew3, zeros_hbm, accp_hbm,
                  src_v, dst_v, ew_v, rows_v, srows_v, sem, acc_sh):
    c = lax.axis_index("c")
    s = lax.axis_index("s")
    wid = s * NC + c

    @pl.when(s == 0)
    def _init():
        pltpu.sync_copy(zeros_hbm, acc_sh)

    plsc.subcore_barrier()
    lane = lax.iota(jnp.int32, 16)

    @pl.loop(0, NCHUNK)
    def _chunk(ci):
        pltpu.sync_copy(src3.at[wid].at[ci], src_v)
        pltpu.sync_copy(dst3.at[wid].at[ci], dst_v)
        pltpu.sync_copy(ew3.at[wid].at[ci], ew_v)
        pltpu.async_copy(g_hbm.at[src_v], rows_v, sem).wait()

        @pl.loop(0, GRP)
        def _grp(gi):
            e0 = gi * 16
            ew16 = ew_v[pl.ds(e0, 16)]
            ridx = e0 + lane
            for col in range(D):
                cidx = jnp.full((16,), col, jnp.int32)
                v = plsc.load_gather(rows_v, [ridx, cidx])
                plsc.store_scatter(srows_v, [ridx, cidx], v * ew16)

        pltpu.sync_copy(srows_v, acc_sh.at[dst_v], add=True)

    plsc.subcore_barrier()

    @pl.when(s == 0)
    def _flush():
        pltpu.sync_copy(acc_sh, accp_hbm.at[c])


_sc_edge = pl.kernel(
    _sc_edge_body,
    out_type=jax.ShapeDtypeStruct((NC, N, D), jnp.float32),
    mesh=_mesh,
    scratch_types=[
        pltpu.VMEM((K,), jnp.int32),
        pltpu.VMEM((K,), jnp.int32),
        pltpu.VMEM((K,), jnp.float32),
        pltpu.VMEM((K, D), jnp.float32),
        pltpu.VMEM((K, D), jnp.float32),
        pltpu.SemaphoreType.DMA,
        pltpu.VMEM_SHARED((N, D), jnp.float32),
    ],
    compiler_params=_sc_params,
)

BR = 512


def _tc_lin_body(x_ref, w_ref, degp_ref, g_ref):
    deg = degp_ref[0, :] + degp_ref[1, :] + 2.0
    dis = lax.rsqrt(deg)
    h = jnp.dot(x_ref[...], w_ref[...], preferred_element_type=jnp.float32)
    g_ref[...] = h * dis[:, None]


def _tc_lin(x, w, degp):
    return pl.pallas_call(
        _tc_lin_body,
        grid=(pl.cdiv(N, BR),),
        in_specs=[
            pl.BlockSpec((BR, D), lambda i: (i, 0)),
            pl.BlockSpec((D, D), lambda i: (0, 0)),
            pl.BlockSpec((2, BR), lambda i: (0, i)),
        ],
        out_specs=pl.BlockSpec((BR, D), lambda i: (i, 0)),
        out_shape=jax.ShapeDtypeStruct((N, D), jnp.float32),
    )(x, w, degp)


def _tc_comb_body(accp_ref, g_ref, degp_ref, b_ref, o_ref):
    deg = degp_ref[0, :] + degp_ref[1, :] + 2.0
    dis = lax.rsqrt(deg)
    t = accp_ref[0] + accp_ref[1] + 2.0 * g_ref[...]
    o_ref[...] = t * dis[:, None] + b_ref[...]


def _tc_comb(accp, g, degp, b2):
    return pl.pallas_call(
        _tc_comb_body,
        grid=(pl.cdiv(N, BR),),
        in_specs=[
            pl.BlockSpec((2, BR, D), lambda i: (0, i, 0)),
            pl.BlockSpec((BR, D), lambda i: (i, 0)),
            pl.BlockSpec((2, BR), lambda i: (0, i)),
            pl.BlockSpec((1, D), lambda i: (0, 0)),
        ],
        out_specs=pl.BlockSpec((BR, D), lambda i: (i, 0)),
        out_shape=jax.ShapeDtypeStruct((N, D), jnp.float32),
    )(accp, g, degp, b2)


@jax.jit
def kernel(x, edge_index, edge_weight, W, b):
    src3 = edge_index[0].astype(jnp.int32).reshape(NW, NCHUNK, K)
    dst3 = edge_index[1].astype(jnp.int32).reshape(NW, NCHUNK, K)
    ew3 = edge_weight.astype(jnp.float32).reshape(NW, NCHUNK, K)
    zeros1 = jnp.zeros((N,), jnp.float32)
    zeros2 = jnp.zeros((N, D), jnp.float32)
    degp = _sc_deg(dst3, ew3, zeros1)
    g = _tc_lin(x, W, degp)
    accp = _sc_edge(g, src3, dst3, ew3, zeros2)
    return _tc_comb(accp, g, degp, b.reshape(1, D))


# speedup vs baseline: 9.5611x; 2.4000x over previous
import jax
import jax.numpy as jnp
from jax import lax
from jax.experimental import pallas as pl
from jax.experimental.pallas import tpu as pltpu
from jax.experimental.pallas import tpu_sc as plsc

N = 10000
E = 320000
D = 128

NC = 2
NS = 16
NW = NC * NS
EPW = E // NW
K = 80
NCHUNK = EPW // K
GRP = K // 16

_mesh = plsc.VectorSubcoreMesh(
    core_axis_name="c", subcore_axis_name="s", num_cores=NC, num_subcores=NS
)
_sc_params = pltpu.CompilerParams(needs_layout_passes=False)


def _sc_deg_body(dst3, ew3, zeros_hbm, degp_hbm, dsts_v, ew_v, sem, deg_sh):
    c = lax.axis_index("c")
    s = lax.axis_index("s")
    wid = s * NC + c

    @pl.when(s == 0)
    def _init():
        pltpu.sync_copy(zeros_hbm, deg_sh)

    pltpu.sync_copy(dst3.at[wid], dsts_v)
    pltpu.sync_copy(ew3.at[wid], ew_v)
    plsc.subcore_barrier()

    @pl.loop(0, NCHUNK)
    def _fire(ci):
        pltpu.async_copy(ew_v.at[ci], deg_sh.at[dsts_v.at[ci]], sem, add=True)

    @pl.loop(0, NCHUNK)
    def _drain(ci):
        pltpu.make_async_copy(ew_v.at[ci], deg_sh.at[dsts_v.at[ci]], sem).wait()

    plsc.subcore_barrier()

    @pl.when(s == 0)
    def _flush():
        pltpu.sync_copy(deg_sh, degp_hbm.at[c])


_sc_deg = pl.kernel(
    _sc_deg_body,
    out_type=jax.ShapeDtypeStruct((NC, N), jnp.float32),
    mesh=_mesh,
    scratch_types=[
        pltpu.VMEM((NCHUNK, K), jnp.int32),
        pltpu.VMEM((NCHUNK, K), jnp.float32),
        pltpu.SemaphoreType.DMA,
        pltpu.VMEM_SHARED((N,), jnp.float32),
    ],
    compiler_params=_sc_params,
)


def _sc_edge_body(g_hbm, src3, dst3, ew3, zeros_hbm, accp_hbm,
                  src_v, dst_v, ew_v, rows_v, srows_v, sem, acc_sh):
    c = lax.axis_index("c")
    s = lax.axis_index("s")
    wid = s * NC + c

    @pl.when(s == 0)
    def _init():
        pltpu.sync_copy(zeros_hbm, acc_sh)

    plsc.subcore_barrier()
    lane = lax.iota(jnp.int32, 16)

    @pl.loop(0, NCHUNK)
    def _chunk(ci):
        pltpu.sync_copy(src3.at[wid].at[ci], src_v)
        pltpu.sync_copy(dst3.at[wid].at[ci], dst_v)
        pltpu.sync_copy(ew3.at[wid].at[ci], ew_v)
        pltpu.async_copy(g_hbm.at[src_v], rows_v, sem).wait()

        @pl.loop(0, K)
        def _edge(e):
            w16 = plsc.load_gather(ew_v, [jnp.full((16,), e, jnp.int32)])
            for p in range(D // 16):
                srows_v[e, pl.ds(p * 16, 16)] = (
                    rows_v[e, pl.ds(p * 16, 16)] * w16)

        pltpu.sync_copy(srows_v, acc_sh.at[dst_v], add=True)

    plsc.subcore_barrier()

    @pl.when(s == 0)
    def _flush():
        pltpu.sync_copy(acc_sh, accp_hbm.at[c])


_sc_edge = pl.kernel(
    _sc_edge_body,
    out_type=jax.ShapeDtypeStruct((NC, N, D), jnp.float32),
    mesh=_mesh,
    scratch_types=[
        pltpu.VMEM((K,), jnp.int32),
        pltpu.VMEM((K,), jnp.int32),
        pltpu.VMEM((K,), jnp.float32),
        pltpu.VMEM((K, D), jnp.float32),
        pltpu.VMEM((K, D), jnp.float32),
        pltpu.SemaphoreType.DMA,
        pltpu.VMEM_SHARED((N, D), jnp.float32),
    ],
    compiler_params=_sc_params,
)

BR = 512


def _tc_lin_body(x_ref, w_ref, degp_ref, g_ref):
    deg = degp_ref[0, :] + degp_ref[1, :] + 2.0
    dis = lax.rsqrt(deg)
    h = jnp.dot(x_ref[...], w_ref[...], preferred_element_type=jnp.float32)
    g_ref[...] = h * dis[:, None]


def _tc_lin(x, w, degp):
    return pl.pallas_call(
        _tc_lin_body,
        grid=(pl.cdiv(N, BR),),
        in_specs=[
            pl.BlockSpec((BR, D), lambda i: (i, 0)),
            pl.BlockSpec((D, D), lambda i: (0, 0)),
            pl.BlockSpec((2, BR), lambda i: (0, i)),
        ],
        out_specs=pl.BlockSpec((BR, D), lambda i: (i, 0)),
        out_shape=jax.ShapeDtypeStruct((N, D), jnp.float32),
    )(x, w, degp)


def _tc_comb_body(accp_ref, g_ref, degp_ref, b_ref, o_ref):
    deg = degp_ref[0, :] + degp_ref[1, :] + 2.0
    dis = lax.rsqrt(deg)
    t = accp_ref[0] + accp_ref[1] + 2.0 * g_ref[...]
    o_ref[...] = t * dis[:, None] + b_ref[...]


def _tc_comb(accp, g, degp, b2):
    return pl.pallas_call(
        _tc_comb_body,
        grid=(pl.cdiv(N, BR),),
        in_specs=[
            pl.BlockSpec((2, BR, D), lambda i: (0, i, 0)),
            pl.BlockSpec((BR, D), lambda i: (i, 0)),
            pl.BlockSpec((2, BR), lambda i: (0, i)),
            pl.BlockSpec((1, D), lambda i: (0, 0)),
        ],
        out_specs=pl.BlockSpec((BR, D), lambda i: (i, 0)),
        out_shape=jax.ShapeDtypeStruct((N, D), jnp.float32),
    )(accp, g, degp, b2)


@jax.jit
def kernel(x, edge_index, edge_weight, W, b):
    src3 = edge_index[0].astype(jnp.int32).reshape(NW, NCHUNK, K)
    dst3 = edge_index[1].astype(jnp.int32).reshape(NW, NCHUNK, K)
    ew3 = edge_weight.astype(jnp.float32).reshape(NW, NCHUNK, K)
    zeros1 = jnp.zeros((N,), jnp.float32)
    zeros2 = jnp.zeros((N, D), jnp.float32)
    degp = _sc_deg(dst3, ew3, zeros1)
    g = _tc_lin(x, W, degp)
    accp = _sc_edge(g, src3, dst3, ew3, zeros2)
    return _tc_comb(accp, g, degp, b.reshape(1, D))
